# baseline (speedup 1.0000x reference)
import functools

import jax
import jax.numpy as jnp
from jax import lax
from jax.experimental import pallas as pl
from jax.experimental.pallas import tpu as pltpu
from jax.experimental.pallas import tpu_sc as plsc

N_NODES = 10000
N_EDGES = 320000
D = 128
DH = D // 2
N_GRAPHS = 64
N_CLASSES = 10

NC = 2
NS = 16
CHUNK = 128
CPT = 160
E_PAD = NS * CPT * CHUNK
NPAD = N_NODES + 112
RPT = NPAD // NS
ROWBLK = 1000
NBLK = N_NODES // ROWBLK


def _mesh():
    return plsc.VectorSubcoreMesh(core_axis_name="c", subcore_axis_name="s")


def _zero_rows(ref, nrows, ncols):
    @pl.loop(0, nrows)
    def _(r):
        @pl.loop(0, ncols // 16)
        def _(j):
            ref[r, pl.ds(j * 16, 16)] = jnp.zeros((16,), jnp.float32)


def _sc_degree(dst3):
    @functools.partial(
        pl.kernel,
        out_type=jax.ShapeDtypeStruct((NC, NPAD, 16), jnp.float32),
        mesh=_mesh(),
        compiler_params=pltpu.CompilerParams(use_tc_tiling_on_sc=False),
        scratch_types=[
            pltpu.VMEM((CPT // 2, CHUNK), jnp.int32),
            pltpu.VMEM((CHUNK, 16), jnp.float32),
            pltpu.VMEM((CHUNK, 16), jnp.float32),
            pltpu.VMEM_SHARED((NPAD, 16), jnp.float32),
        ],
    )
    def k(dst_hbm, out_hbm, dst_v, ones_v, zero_v, acc_sh):
        cid = lax.axis_index("c")
        sid = lax.axis_index("s")
        wid = cid * NS + sid

        @pl.loop(0, CHUNK)
        def _(r):
            ones_v[r, pl.ds(0, 16)] = jnp.ones((16,), jnp.float32)
            zero_v[r, pl.ds(0, 16)] = jnp.zeros((16,), jnp.float32)

        @pl.loop(0, 4)
        def _(t):
            pltpu.sync_copy(zero_v, acc_sh.at[pl.ds(sid * RPT + t * CHUNK, CHUNK)])
        pltpu.sync_copy(zero_v.at[pl.ds(0, RPT - 4 * CHUNK)],
                        acc_sh.at[pl.ds(sid * RPT + 4 * CHUNK, RPT - 4 * CHUNK)])
        plsc.subcore_barrier()

        pltpu.sync_copy(dst_hbm.at[wid], dst_v)

        @pl.loop(0, CPT // 2)
        def _(c):
            pltpu.sync_copy(ones_v, acc_sh.at[dst_v.at[c]], add=True)

        plsc.subcore_barrier()
        pltpu.sync_copy(acc_sh.at[pl.ds(sid * RPT, RPT)],
                        out_hbm.at[cid].at[pl.ds(sid * RPT, RPT)])

    return k(dst3)


def _sc_aggregate(gsplit, src3, dst3):
    @functools.partial(
        pl.kernel,
        out_type=jax.ShapeDtypeStruct((NC, NPAD, DH), jnp.float32),
        mesh=_mesh(),
        compiler_params=pltpu.CompilerParams(use_tc_tiling_on_sc=False),
        scratch_types=[
            pltpu.VMEM((CPT, CHUNK), jnp.int32),
            pltpu.VMEM((CPT, CHUNK), jnp.int32),
            pltpu.VMEM((4, CHUNK, DH), jnp.float32),
            pltpu.VMEM((CHUNK, DH), jnp.float32),
            pltpu.VMEM_SHARED((NPAD, DH), jnp.float32),
            pltpu.SemaphoreType.DMA,
            pltpu.SemaphoreType.DMA,
            pltpu.SemaphoreType.DMA,
            pltpu.SemaphoreType.DMA,
            pltpu.SemaphoreType.DMA,
            pltpu.SemaphoreType.DMA,
            pltpu.SemaphoreType.DMA,
            pltpu.SemaphoreType.DMA,
        ],
    )
    def k(g_hbm, src_hbm, dst_hbm, out_hbm, src_v, dst_v, rows_v, zero_v,
          acc_sh, g0, g1, g2, g3, s0, s1, s2, s3):
        cid = lax.axis_index("c")
        sid = lax.axis_index("s")
        semg = [g0, g1, g2, g3]
        sems = [s0, s1, s2, s3]
        rows = [rows_v.at[j] for j in range(4)]

        pltpu.sync_copy(src_hbm.at[sid], src_v)
        pltpu.sync_copy(dst_hbm.at[sid], dst_v)

        _zero_rows(zero_v, CHUNK, DH)

        @pl.loop(0, 4)
        def _(t):
            pltpu.sync_copy(zero_v, acc_sh.at[pl.ds(sid * RPT + t * CHUNK, CHUNK)])
        pltpu.sync_copy(zero_v.at[pl.ds(0, RPT - 4 * CHUNK)],
                        acc_sh.at[pl.ds(sid * RPT + 4 * CHUNK, RPT - 4 * CHUNK)])
        plsc.subcore_barrier()

        gsrc = g_hbm.at[cid]
        rows0, rows1 = rows[0], rows[1]
        sem0, sem1 = semg[0], semg[1]
        pltpu.async_copy(gsrc.at[src_v.at[0]], rows0, sem0)

        @pl.loop(0, CPT // 2)
        def _(p):
            c0 = 2 * p
            pltpu.make_async_copy(gsrc.at[src_v.at[0]], rows0, sem0).wait()
            pltpu.async_copy(gsrc.at[src_v.at[c0 + 1]], rows1, sem1)
            pltpu.sync_copy(rows0, acc_sh.at[dst_v.at[c0]], add=True)
            pltpu.make_async_copy(gsrc.at[src_v.at[0]], rows1, sem1).wait()

            @pl.when(p < CPT // 2 - 1)
            def _():
                pltpu.async_copy(gsrc.at[src_v.at[c0 + 2]], rows0, sem0)

            pltpu.sync_copy(rows1, acc_sh.at[dst_v.at[c0 + 1]], add=True)

        plsc.subcore_barrier()
        pltpu.sync_copy(acc_sh.at[pl.ds(sid * RPT, RPT)],
                        out_hbm.at[cid].at[pl.ds(sid * RPT, RPT)])

    return k(gsplit, src3, dst3)


_DOT = (((1,), (0,)), ((), ()))


def _split(h):
    return jnp.stack([h[:, :DH], h[:, DH:]])


def _unsplit(blk):
    return jnp.concatenate([blk[0], blk[1]], axis=-1)


def _tc_layer1(x, dega, degb, W1):
    def body(x_ref, da_ref, db_ref, w_ref, g_ref):
        deg = da_ref[:, 0:1] + db_ref[:, 0:1] + 1.0
        dinv = lax.rsqrt(deg)
        h = lax.dot_general(x_ref[...], w_ref[...], _DOT,
                            precision=lax.Precision.HIGHEST)
        g_ref[...] = _split(h * dinv)

    return pl.pallas_call(
        body,
        grid=(NBLK,),
        in_specs=[
            pl.BlockSpec((ROWBLK, D), lambda i: (i, 0)),
            pl.BlockSpec((ROWBLK, 16), lambda i: (i, 0)),
            pl.BlockSpec((ROWBLK, 16), lambda i: (i, 0)),
            pl.BlockSpec((D, D), lambda i: (0, 0)),
        ],
        out_specs=pl.BlockSpec((NC, ROWBLK, DH), lambda i: (0, i, 0)),
        out_shape=jax.ShapeDtypeStruct((NC, N_NODES, DH), jnp.float32),
    )(x, dega, degb, W1)


def _tc_layer2(acc, g1, dega, degb, b1, W2):
    def body(a_ref, g_ref, da_ref, db_ref, b_ref, w_ref, o_ref):
        deg = da_ref[:, 0:1] + db_ref[:, 0:1] + 1.0
        dinv = lax.rsqrt(deg)
        z = dinv * (_unsplit(a_ref[...]) + _unsplit(g_ref[...])) + b_ref[...]
        h = jnp.maximum(z, 0.0)
        h2 = lax.dot_general(h, w_ref[...], _DOT,
                             precision=lax.Precision.HIGHEST)
        o_ref[...] = _split(h2 * dinv)

    return pl.pallas_call(
        body,
        grid=(NBLK,),
        in_specs=[
            pl.BlockSpec((NC, ROWBLK, DH), lambda i: (0, i, 0)),
            pl.BlockSpec((NC, ROWBLK, DH), lambda i: (0, i, 0)),
            pl.BlockSpec((ROWBLK, 16), lambda i: (i, 0)),
            pl.BlockSpec((ROWBLK, 16), lambda i: (i, 0)),
            pl.BlockSpec((1, D), lambda i: (0, 0)),
            pl.BlockSpec((D, D), lambda i: (0, 0)),
        ],
        out_specs=pl.BlockSpec((NC, ROWBLK, DH), lambda i: (0, i, 0)),
        out_shape=jax.ShapeDtypeStruct((NC, N_NODES, DH), jnp.float32),
    )(acc, g1, dega, degb, b1, W2)


def _tc_final(acc, g2, dega, degb, b2, batf, wl, bl):
    def body(a_ref, g_ref, da_ref, db_ref, b_ref, bat_ref, wl_ref,
             bl_ref, o_ref, pool_ref):
        i = pl.program_id(0)
        deg = da_ref[:, 0:1] + db_ref[:, 0:1] + 1.0
        dinv = lax.rsqrt(deg)
        z = dinv * (_unsplit(a_ref[...]) + _unsplit(g_ref[...])) + b_ref[...]
        h = jnp.maximum(z, 0.0)
        bvec = jnp.reshape(bat_ref[0, 0, :], (1, ROWBLK))
        gids = lax.broadcasted_iota(jnp.int32, (N_GRAPHS, ROWBLK), 0)
        m = (bvec == gids).astype(jnp.float32)
        pm = lax.dot_general(m, h, _DOT, precision=lax.Precision.HIGHEST)

        @pl.when(i == 0)
        def _():
            pool_ref[...] = pm

        @pl.when(i > 0)
        def _():
            pool_ref[...] += pm

        @pl.when(i == NBLK - 1)
        def _():
            o_ref[...] = lax.dot_general(pool_ref[...], wl_ref[...], _DOT,
                                         precision=lax.Precision.HIGHEST) + bl_ref[...]

    return pl.pallas_call(
        body,
        grid=(NBLK,),
        in_specs=[
            pl.BlockSpec((NC, ROWBLK, DH), lambda i: (0, i, 0)),
            pl.BlockSpec((NC, ROWBLK, DH), lambda i: (0, i, 0)),
            pl.BlockSpec((ROWBLK, 16), lambda i: (i, 0)),
            pl.BlockSpec((ROWBLK, 16), lambda i: (i, 0)),
            pl.BlockSpec((1, D), lambda i: (0, 0)),
            pl.BlockSpec((1, 1, ROWBLK), lambda i: (i, 0, 0)),
            pl.BlockSpec((D, D), lambda i: (0, 0)),
            pl.BlockSpec((1, D), lambda i: (0, 0)),
        ],
        out_specs=pl.BlockSpec((N_GRAPHS, D), lambda i: (0, 0)),
        out_shape=jax.ShapeDtypeStruct((N_GRAPHS, D), jnp.float32),
        scratch_shapes=[pltpu.VMEM((N_GRAPHS, D), jnp.float32)],
    )(acc, g2, dega, degb, b2, batf, wl, bl)


def kernel(x, edge_index, batch, W1, b1, W2, b2, Wlin, blin):
    src = edge_index[0].astype(jnp.int32)
    dst = edge_index[1].astype(jnp.int32)
    pad = E_PAD - N_EDGES
    src_p = jnp.concatenate([src, jnp.zeros((pad,), jnp.int32)])
    dst_p = jnp.concatenate([dst, jnp.full((pad,), N_NODES, jnp.int32)])
    src3 = src_p.reshape(NS, CPT, CHUNK)
    dst3 = dst_p.reshape(NS, CPT, CHUNK)
    dst3_32 = dst_p.reshape(NC * NS, CPT // 2, CHUNK)

    deg2 = _sc_degree(dst3_32)
    dega, degb = deg2[0], deg2[1]

    g1 = _tc_layer1(x, dega, degb, W1)
    acc1 = _sc_aggregate(g1, src3, dst3)
    g2 = _tc_layer2(acc1, g1, dega, degb, jnp.reshape(b1, (1, D)), W2)
    acc2 = _sc_aggregate(g2, src3, dst3)

    batf = batch.astype(jnp.int32).reshape(NBLK, 1, ROWBLK)
    wl = jnp.zeros((D, D), jnp.float32).at[:, :N_CLASSES].set(Wlin)
    bl = jnp.zeros((1, D), jnp.float32).at[0, :N_CLASSES].set(blin)
    outp = _tc_final(acc2, g2, dega, degb,
                     jnp.reshape(b2, (1, D)), batf, wl, bl)
    return outp[:, :N_CLASSES]


# speedup vs baseline: 1.0014x; 1.0014x over previous
import functools

import jax
import jax.numpy as jnp
from jax import lax
from jax.experimental import pallas as pl
from jax.experimental.pallas import tpu as pltpu
from jax.experimental.pallas import tpu_sc as plsc

N_NODES = 10000
N_EDGES = 320000
D = 128
DH = D // 2
N_GRAPHS = 64
N_CLASSES = 10

NC = 2
NS = 16
CHUNK = 128
CPT = 160
E_PAD = NS * CPT * CHUNK
NPAD = N_NODES + 112
RPT = NPAD // NS
ROWBLK = 1000
NBLK = N_NODES // ROWBLK


def _mesh():
    return plsc.VectorSubcoreMesh(core_axis_name="c", subcore_axis_name="s")


def _zero_rows(ref, nrows, ncols):
    @pl.loop(0, nrows)
    def _(r):
        @pl.loop(0, ncols // 16)
        def _(j):
            ref[r, pl.ds(j * 16, 16)] = jnp.zeros((16,), jnp.float32)


def _sc_degree(dst3):
    @functools.partial(
        pl.kernel,
        out_type=jax.ShapeDtypeStruct((NC, NPAD, 16), jnp.float32),
        mesh=_mesh(),
        compiler_params=pltpu.CompilerParams(use_tc_tiling_on_sc=False),
        scratch_types=[
            pltpu.VMEM((CPT // 2, CHUNK), jnp.int32),
            pltpu.VMEM((CHUNK, 16), jnp.float32),
            pltpu.VMEM((CHUNK, 16), jnp.float32),
            pltpu.VMEM_SHARED((NPAD, 16), jnp.float32),
        ],
    )
    def k(dst_hbm, out_hbm, dst_v, ones_v, zero_v, acc_sh):
        cid = lax.axis_index("c")
        sid = lax.axis_index("s")
        wid = cid * NS + sid

        @pl.loop(0, CHUNK)
        def _(r):
            ones_v[r, pl.ds(0, 16)] = jnp.ones((16,), jnp.float32)
            zero_v[r, pl.ds(0, 16)] = jnp.zeros((16,), jnp.float32)

        @pl.loop(0, 4)
        def _(t):
            pltpu.sync_copy(zero_v, acc_sh.at[pl.ds(sid * RPT + t * CHUNK, CHUNK)])
        pltpu.sync_copy(zero_v.at[pl.ds(0, RPT - 4 * CHUNK)],
                        acc_sh.at[pl.ds(sid * RPT + 4 * CHUNK, RPT - 4 * CHUNK)])
        plsc.subcore_barrier()

        pltpu.sync_copy(dst_hbm.at[wid], dst_v)

        @pl.loop(0, CPT // 2)
        def _(c):
            pltpu.sync_copy(ones_v, acc_sh.at[dst_v.at[c]], add=True)

        plsc.subcore_barrier()
        pltpu.sync_copy(acc_sh.at[pl.ds(sid * RPT, RPT)],
                        out_hbm.at[cid].at[pl.ds(sid * RPT, RPT)])

    return k(dst3)


def _sc_aggregate(gsplit, src3, dst3):
    @functools.partial(
        pl.kernel,
        out_type=jax.ShapeDtypeStruct((NC, NPAD, DH), jnp.float32),
        mesh=_mesh(),
        compiler_params=pltpu.CompilerParams(use_tc_tiling_on_sc=False),
        scratch_types=[
            pltpu.VMEM((CPT, CHUNK), jnp.int32),
            pltpu.VMEM((CPT, CHUNK), jnp.int32),
            pltpu.VMEM((CHUNK, DH), jnp.float32),
            pltpu.VMEM((CHUNK, DH), jnp.float32),
            pltpu.VMEM((CHUNK, DH), jnp.float32),
            pltpu.VMEM_SHARED((NPAD, DH), jnp.float32),
            pltpu.SemaphoreType.DMA,
            pltpu.SemaphoreType.DMA,
        ],
    )
    def k(g_hbm, src_hbm, dst_hbm, out_hbm, src_v, dst_v, rows0, rows1,
          zero_v, acc_sh, sem0, sem1):
        cid = lax.axis_index("c")
        sid = lax.axis_index("s")

        pltpu.sync_copy(src_hbm.at[sid], src_v)
        pltpu.sync_copy(dst_hbm.at[sid], dst_v)

        _zero_rows(zero_v, CHUNK, DH)

        @pl.loop(0, 4)
        def _(t):
            pltpu.sync_copy(zero_v, acc_sh.at[pl.ds(sid * RPT + t * CHUNK, CHUNK)])
        pltpu.sync_copy(zero_v.at[pl.ds(0, RPT - 4 * CHUNK)],
                        acc_sh.at[pl.ds(sid * RPT + 4 * CHUNK, RPT - 4 * CHUNK)])
        plsc.subcore_barrier()

        gsrc = g_hbm.at[cid]
        pltpu.async_copy(gsrc.at[src_v.at[0]], rows0, sem0)

        @pl.loop(0, CPT // 2)
        def _(p):
            c0 = 2 * p
            pltpu.make_async_copy(gsrc.at[src_v.at[0]], rows0, sem0).wait()
            pltpu.async_copy(gsrc.at[src_v.at[c0 + 1]], rows1, sem1)
            pltpu.sync_copy(rows0, acc_sh.at[dst_v.at[c0]], add=True)
            pltpu.make_async_copy(gsrc.at[src_v.at[0]], rows1, sem1).wait()

            @pl.when(p < CPT // 2 - 1)
            def _():
                pltpu.async_copy(gsrc.at[src_v.at[c0 + 2]], rows0, sem0)

            pltpu.sync_copy(rows1, acc_sh.at[dst_v.at[c0 + 1]], add=True)

        plsc.subcore_barrier()
        pltpu.sync_copy(acc_sh.at[pl.ds(sid * RPT, RPT)],
                        out_hbm.at[cid].at[pl.ds(sid * RPT, RPT)])

    return k(gsplit, src3, dst3)


_DOT = (((1,), (0,)), ((), ()))


def _split(h):
    return jnp.stack([h[:, :DH], h[:, DH:]])


def _unsplit(blk):
    return jnp.concatenate([blk[0], blk[1]], axis=-1)


def _tc_layer1(x, dega, degb, W1):
    def body(x_ref, da_ref, db_ref, w_ref, g_ref):
        deg = da_ref[:, 0:1] + db_ref[:, 0:1] + 1.0
        dinv = lax.rsqrt(deg)
        h = lax.dot_general(x_ref[...], w_ref[...], _DOT,
                            precision=lax.Precision.HIGHEST)
        g_ref[...] = _split(h * dinv)

    return pl.pallas_call(
        body,
        grid=(NBLK,),
        in_specs=[
            pl.BlockSpec((ROWBLK, D), lambda i: (i, 0)),
            pl.BlockSpec((ROWBLK, 16), lambda i: (i, 0)),
            pl.BlockSpec((ROWBLK, 16), lambda i: (i, 0)),
            pl.BlockSpec((D, D), lambda i: (0, 0)),
        ],
        out_specs=pl.BlockSpec((NC, ROWBLK, DH), lambda i: (0, i, 0)),
        out_shape=jax.ShapeDtypeStruct((NC, N_NODES, DH), jnp.float32),
    )(x, dega, degb, W1)


def _tc_layer2(acc, g1, dega, degb, b1, W2):
    def body(a_ref, g_ref, da_ref, db_ref, b_ref, w_ref, o_ref):
        deg = da_ref[:, 0:1] + db_ref[:, 0:1] + 1.0
        dinv = lax.rsqrt(deg)
        z = dinv * (_unsplit(a_ref[...]) + _unsplit(g_ref[...])) + b_ref[...]
        h = jnp.maximum(z, 0.0)
        h2 = lax.dot_general(h, w_ref[...], _DOT,
                             precision=lax.Precision.HIGHEST)
        o_ref[...] = _split(h2 * dinv)

    return pl.pallas_call(
        body,
        grid=(NBLK,),
        in_specs=[
            pl.BlockSpec((NC, ROWBLK, DH), lambda i: (0, i, 0)),
            pl.BlockSpec((NC, ROWBLK, DH), lambda i: (0, i, 0)),
            pl.BlockSpec((ROWBLK, 16), lambda i: (i, 0)),
            pl.BlockSpec((ROWBLK, 16), lambda i: (i, 0)),
            pl.BlockSpec((1, D), lambda i: (0, 0)),
            pl.BlockSpec((D, D), lambda i: (0, 0)),
        ],
        out_specs=pl.BlockSpec((NC, ROWBLK, DH), lambda i: (0, i, 0)),
        out_shape=jax.ShapeDtypeStruct((NC, N_NODES, DH), jnp.float32),
    )(acc, g1, dega, degb, b1, W2)


def _tc_final(acc, g2, dega, degb, b2, batf, wl, bl):
    def body(a_ref, g_ref, da_ref, db_ref, b_ref, bat_ref, wl_ref,
             bl_ref, o_ref, pool_ref):
        i = pl.program_id(0)
        deg = da_ref[:, 0:1] + db_ref[:, 0:1] + 1.0
        dinv = lax.rsqrt(deg)
        z = dinv * (_unsplit(a_ref[...]) + _unsplit(g_ref[...])) + b_ref[...]
        h = jnp.maximum(z, 0.0)
        bvec = jnp.reshape(bat_ref[0, 0, :], (1, ROWBLK))
        gids = lax.broadcasted_iota(jnp.int32, (N_GRAPHS, ROWBLK), 0)
        m = (bvec == gids).astype(jnp.float32)
        pm = lax.dot_general(m, h, _DOT, precision=lax.Precision.HIGHEST)

        @pl.when(i == 0)
        def _():
            pool_ref[...] = pm

        @pl.when(i > 0)
        def _():
            pool_ref[...] += pm

        @pl.when(i == NBLK - 1)
        def _():
            o_ref[...] = lax.dot_general(pool_ref[...], wl_ref[...], _DOT,
                                         precision=lax.Precision.HIGHEST) + bl_ref[...]

    return pl.pallas_call(
        body,
        grid=(NBLK,),
        in_specs=[
            pl.BlockSpec((NC, ROWBLK, DH), lambda i: (0, i, 0)),
            pl.BlockSpec((NC, ROWBLK, DH), lambda i: (0, i, 0)),
            pl.BlockSpec((ROWBLK, 16), lambda i: (i, 0)),
            pl.BlockSpec((ROWBLK, 16), lambda i: (i, 0)),
            pl.BlockSpec((1, D), lambda i: (0, 0)),
            pl.BlockSpec((1, 1, ROWBLK), lambda i: (i, 0, 0)),
            pl.BlockSpec((D, D), lambda i: (0, 0)),
            pl.BlockSpec((1, D), lambda i: (0, 0)),
        ],
        out_specs=pl.BlockSpec((N_GRAPHS, D), lambda i: (0, 0)),
        out_shape=jax.ShapeDtypeStruct((N_GRAPHS, D), jnp.float32),
        scratch_shapes=[pltpu.VMEM((N_GRAPHS, D), jnp.float32)],
    )(acc, g2, dega, degb, b2, batf, wl, bl)


def kernel(x, edge_index, batch, W1, b1, W2, b2, Wlin, blin):
    src = edge_index[0].astype(jnp.int32)
    dst = edge_index[1].astype(jnp.int32)
    pad = E_PAD - N_EDGES
    src_p = jnp.concatenate([src, jnp.zeros((pad,), jnp.int32)])
    dst_p = jnp.concatenate([dst, jnp.full((pad,), N_NODES, jnp.int32)])
    src3 = src_p.reshape(NS, CPT, CHUNK)
    dst3 = dst_p.reshape(NS, CPT, CHUNK)
    dst3_32 = dst_p.reshape(NC * NS, CPT // 2, CHUNK)

    deg2 = _sc_degree(dst3_32)
    dega, degb = deg2[0], deg2[1]

    g1 = _tc_layer1(x, dega, degb, W1)
    acc1 = _sc_aggregate(g1, src3, dst3)
    g2 = _tc_layer2(acc1, g1, dega, degb, jnp.reshape(b1, (1, D)), W2)
    acc2 = _sc_aggregate(g2, src3, dst3)

    batf = batch.astype(jnp.int32).reshape(NBLK, 1, ROWBLK)
    wl = jnp.zeros((D, D), jnp.float32).at[:, :N_CLASSES].set(Wlin)
    bl = jnp.zeros((1, D), jnp.float32).at[0, :N_CLASSES].set(blin)
    outp = _tc_final(acc2, g2, dega, degb,
                     jnp.reshape(b2, (1, D)), batf, wl, bl)
    return outp[:, :N_CLASSES]


# speedup vs baseline: 1.7581x; 1.7557x over previous
import functools

import jax
import jax.numpy as jnp
from jax import lax
from jax.experimental import pallas as pl
from jax.experimental.pallas import tpu as pltpu
from jax.experimental.pallas import tpu_sc as plsc

N_NODES = 10000
N_EDGES = 320000
D = 128
DH = D // 2
N_GRAPHS = 64
N_CLASSES = 10

NC = 2
NS = 16
CHUNK = 128
CPT = 160
E_PAD = NS * CPT * CHUNK
NPAD = N_NODES + 112
RPT = NPAD // NS
ROWBLK = 1000
NBLK = N_NODES // ROWBLK


def _mesh():
    return plsc.VectorSubcoreMesh(core_axis_name="c", subcore_axis_name="s")


def _zero_rows(ref, nrows, ncols):
    @pl.loop(0, nrows)
    def _(r):
        @pl.loop(0, ncols // 16)
        def _(j):
            ref[r, pl.ds(j * 16, 16)] = jnp.zeros((16,), jnp.float32)


def _sc_degree(dst3):
    @functools.partial(
        pl.kernel,
        out_type=jax.ShapeDtypeStruct((NC, NPAD, 16), jnp.float32),
        mesh=_mesh(),
        compiler_params=pltpu.CompilerParams(use_tc_tiling_on_sc=False),
        scratch_types=[
            pltpu.VMEM((CPT // 2, CHUNK), jnp.int32),
            pltpu.VMEM((CHUNK, 16), jnp.float32),
            pltpu.VMEM((CHUNK, 16), jnp.float32),
            pltpu.VMEM_SHARED((NPAD, 16), jnp.float32),
        ],
    )
    def k(dst_hbm, out_hbm, dst_v, ones_v, zero_v, acc_sh):
        cid = lax.axis_index("c")
        sid = lax.axis_index("s")
        wid = cid * NS + sid

        @pl.loop(0, CHUNK)
        def _(r):
            ones_v[r, pl.ds(0, 16)] = jnp.ones((16,), jnp.float32)
            zero_v[r, pl.ds(0, 16)] = jnp.zeros((16,), jnp.float32)

        @pl.loop(0, 4)
        def _(t):
            pltpu.sync_copy(zero_v, acc_sh.at[pl.ds(sid * RPT + t * CHUNK, CHUNK)])
        pltpu.sync_copy(zero_v.at[pl.ds(0, RPT - 4 * CHUNK)],
                        acc_sh.at[pl.ds(sid * RPT + 4 * CHUNK, RPT - 4 * CHUNK)])
        plsc.subcore_barrier()

        pltpu.sync_copy(dst_hbm.at[wid], dst_v)

        @pl.loop(0, CPT // 2)
        def _(c):
            pltpu.sync_copy(ones_v, acc_sh.at[dst_v.at[c]], add=True)

        plsc.subcore_barrier()
        pltpu.sync_copy(acc_sh.at[pl.ds(sid * RPT, RPT)],
                        out_hbm.at[cid].at[pl.ds(sid * RPT, RPT)])

    return k(dst3)


def _sc_aggregate(gsplit, src3, dst3):
    @functools.partial(
        pl.kernel,
        out_type=jax.ShapeDtypeStruct((NC, NPAD, DH), jnp.float32),
        mesh=_mesh(),
        compiler_params=pltpu.CompilerParams(use_tc_tiling_on_sc=False),
        scratch_types=[
            pltpu.VMEM((CPT, CHUNK), jnp.int32),
            pltpu.VMEM((CPT, CHUNK), jnp.int32),
            pltpu.VMEM((CHUNK, DH), jnp.float32),
            pltpu.VMEM((CHUNK, DH), jnp.float32),
            pltpu.VMEM((CHUNK, DH), jnp.float32),
            pltpu.VMEM_SHARED((NPAD, DH), jnp.float32),
            pltpu.SemaphoreType.DMA,
            pltpu.SemaphoreType.DMA,
        ],
    )
    def k(g_hbm, src_hbm, dst_hbm, out_hbm, src_v, dst_v, rows0, rows1,
          zero_v, acc_sh, sem0, sem1):
        cid = lax.axis_index("c")
        sid = lax.axis_index("s")

        pltpu.sync_copy(src_hbm.at[sid], src_v)
        pltpu.sync_copy(dst_hbm.at[sid], dst_v)

        _zero_rows(zero_v, CHUNK, DH)

        @pl.loop(0, 4)
        def _(t):
            pltpu.sync_copy(zero_v, acc_sh.at[pl.ds(sid * RPT + t * CHUNK, CHUNK)])
        pltpu.sync_copy(zero_v.at[pl.ds(0, RPT - 4 * CHUNK)],
                        acc_sh.at[pl.ds(sid * RPT + 4 * CHUNK, RPT - 4 * CHUNK)])
        plsc.subcore_barrier()

        gsrc = g_hbm.at[cid]
        pltpu.async_copy(gsrc.at[src_v.at[0]], rows0, sem0)

        @pl.loop(0, CPT // 2)
        def _(p):
            c0 = 2 * p
            pltpu.make_async_copy(gsrc.at[src_v.at[0]], rows0, sem0).wait()
            pltpu.async_copy(gsrc.at[src_v.at[c0 + 1]], rows1, sem1)
            pltpu.sync_copy(rows0, acc_sh.at[dst_v.at[c0]], add=True)
            pltpu.make_async_copy(gsrc.at[src_v.at[0]], rows1, sem1).wait()

            @pl.when(p < CPT // 2 - 1)
            def _():
                pltpu.async_copy(gsrc.at[src_v.at[c0 + 2]], rows0, sem0)

            pltpu.sync_copy(rows1, acc_sh.at[dst_v.at[c0 + 1]], add=True)

        plsc.subcore_barrier()
        pltpu.sync_copy(acc_sh.at[pl.ds(sid * RPT, RPT)],
                        out_hbm.at[cid].at[pl.ds(sid * RPT, RPT)])

    return k(gsplit, src3, dst3)


_DOT = (((1,), (0,)), ((), ()))


def _split(h):
    return jnp.stack([h[:, :DH], h[:, DH:]])


def _unsplit(blk):
    return jnp.concatenate([blk[0], blk[1]], axis=-1)


def _tc_layer1(x, dega, degb, W1):
    def body(x_ref, da_ref, db_ref, w_ref, g_ref):
        deg = da_ref[:, 0:1] + db_ref[:, 0:1] + 1.0
        dinv = lax.rsqrt(deg)
        h = lax.dot_general(x_ref[...], w_ref[...], _DOT,
                            precision=lax.Precision.HIGHEST)
        g_ref[...] = _split(h * dinv)

    return pl.pallas_call(
        body,
        grid=(NBLK,),
        in_specs=[
            pl.BlockSpec((ROWBLK, D), lambda i: (i, 0)),
            pl.BlockSpec((ROWBLK, 16), lambda i: (i, 0)),
            pl.BlockSpec((ROWBLK, 16), lambda i: (i, 0)),
            pl.BlockSpec((D, D), lambda i: (0, 0)),
        ],
        out_specs=pl.BlockSpec((NC, ROWBLK, DH), lambda i: (0, i, 0)),
        out_shape=jax.ShapeDtypeStruct((NC, N_NODES, DH), jnp.float32),
    )(x, dega, degb, W1)


def _tc_layer2(acc, g1, dega, degb, b1, W2):
    def body(a_ref, g_ref, da_ref, db_ref, b_ref, w_ref, o_ref):
        deg = da_ref[:, 0:1] + db_ref[:, 0:1] + 1.0
        dinv = lax.rsqrt(deg)
        z = dinv * (_unsplit(a_ref[...]) + _unsplit(g_ref[...])) + b_ref[...]
        h = jnp.maximum(z, 0.0)
        h2 = lax.dot_general(h, w_ref[...], _DOT,
                             precision=lax.Precision.HIGHEST)
        o_ref[...] = _split(h2 * dinv)

    return pl.pallas_call(
        body,
        grid=(NBLK,),
        in_specs=[
            pl.BlockSpec((NC, ROWBLK, DH), lambda i: (0, i, 0)),
            pl.BlockSpec((NC, ROWBLK, DH), lambda i: (0, i, 0)),
            pl.BlockSpec((ROWBLK, 16), lambda i: (i, 0)),
            pl.BlockSpec((ROWBLK, 16), lambda i: (i, 0)),
            pl.BlockSpec((1, D), lambda i: (0, 0)),
            pl.BlockSpec((D, D), lambda i: (0, 0)),
        ],
        out_specs=pl.BlockSpec((NC, ROWBLK, DH), lambda i: (0, i, 0)),
        out_shape=jax.ShapeDtypeStruct((NC, N_NODES, DH), jnp.float32),
    )(acc, g1, dega, degb, b1, W2)


def _tc_final(acc, g2, dega, degb, b2, batf, wl, bl):
    def body(a_ref, g_ref, da_ref, db_ref, b_ref, bat_ref, wl_ref,
             bl_ref, o_ref, pool_ref):
        i = pl.program_id(0)
        deg = da_ref[:, 0:1] + db_ref[:, 0:1] + 1.0
        dinv = lax.rsqrt(deg)
        z = dinv * (_unsplit(a_ref[...]) + _unsplit(g_ref[...])) + b_ref[...]
        h = jnp.maximum(z, 0.0)
        bvec = jnp.reshape(bat_ref[0, 0, :], (1, ROWBLK))
        gids = lax.broadcasted_iota(jnp.int32, (N_GRAPHS, ROWBLK), 0)
        m = (bvec == gids).astype(jnp.float32)
        pm = lax.dot_general(m, h, _DOT, precision=lax.Precision.HIGHEST)

        @pl.when(i == 0)
        def _():
            pool_ref[...] = pm

        @pl.when(i > 0)
        def _():
            pool_ref[...] += pm

        @pl.when(i == NBLK - 1)
        def _():
            o_ref[...] = lax.dot_general(pool_ref[...], wl_ref[...], _DOT,
                                         precision=lax.Precision.HIGHEST) + bl_ref[...]

    return pl.pallas_call(
        body,
        grid=(NBLK,),
        in_specs=[
            pl.BlockSpec((NC, ROWBLK, DH), lambda i: (0, i, 0)),
            pl.BlockSpec((NC, ROWBLK, DH), lambda i: (0, i, 0)),
            pl.BlockSpec((ROWBLK, 16), lambda i: (i, 0)),
            pl.BlockSpec((ROWBLK, 16), lambda i: (i, 0)),
            pl.BlockSpec((1, D), lambda i: (0, 0)),
            pl.BlockSpec((1, 1, ROWBLK), lambda i: (i, 0, 0)),
            pl.BlockSpec((D, D), lambda i: (0, 0)),
            pl.BlockSpec((1, D), lambda i: (0, 0)),
        ],
        out_specs=pl.BlockSpec((N_GRAPHS, D), lambda i: (0, 0)),
        out_shape=jax.ShapeDtypeStruct((N_GRAPHS, D), jnp.float32),
        scratch_shapes=[pltpu.VMEM((N_GRAPHS, D), jnp.float32)],
    )(acc, g2, dega, degb, b2, batf, wl, bl)


def kernel(x, edge_index, batch, W1, b1, W2, b2, Wlin, blin):
    src = edge_index[0].astype(jnp.int32)
    dst = edge_index[1].astype(jnp.int32)
    pad = E_PAD - N_EDGES
    pad_i = jnp.arange(pad, dtype=jnp.int32)
    src_p = jnp.concatenate([src, pad_i % N_NODES])
    dst_p = jnp.concatenate([dst, N_NODES + pad_i % (NPAD - N_NODES)])
    src3 = src_p.reshape(NS, CPT, CHUNK)
    dst3 = dst_p.reshape(NS, CPT, CHUNK)
    dst3_32 = dst_p.reshape(NC * NS, CPT // 2, CHUNK)

    deg2 = _sc_degree(dst3_32)
    dega, degb = deg2[0], deg2[1]

    g1 = _tc_layer1(x, dega, degb, W1)
    acc1 = _sc_aggregate(g1, src3, dst3)
    g2 = _tc_layer2(acc1, g1, dega, degb, jnp.reshape(b1, (1, D)), W2)
    acc2 = _sc_aggregate(g2, src3, dst3)

    batf = batch.astype(jnp.int32).reshape(NBLK, 1, ROWBLK)
    wl = jnp.zeros((D, D), jnp.float32).at[:, :N_CLASSES].set(Wlin)
    bl = jnp.zeros((1, D), jnp.float32).at[0, :N_CLASSES].set(blin)
    outp = _tc_final(acc2, g2, dega, degb,
                     jnp.reshape(b2, (1, D)), batf, wl, bl)
    return outp[:, :N_CLASSES]


# speedup vs baseline: 2.1341x; 1.2139x over previous
import functools

import jax
import jax.numpy as jnp
from jax import lax
from jax.experimental import pallas as pl
from jax.experimental.pallas import tpu as pltpu
from jax.experimental.pallas import tpu_sc as plsc

N_NODES = 10000
N_EDGES = 320000
D = 128
DH = D // 2
N_GRAPHS = 64
N_CLASSES = 10

NC = 2
NS = 16
CHUNK = 128
CPT = 160
E_PAD = NS * CPT * CHUNK
NPAD = N_NODES + 112
RPT = NPAD // NS
ROWBLK = 1000
NBLK = N_NODES // ROWBLK


def _mesh():
    return plsc.VectorSubcoreMesh(core_axis_name="c", subcore_axis_name="s")


def _zero_rows(ref, nrows, ncols):
    @pl.loop(0, nrows)
    def _(r):
        @pl.loop(0, ncols // 16)
        def _(j):
            ref[r, pl.ds(j * 16, 16)] = jnp.zeros((16,), jnp.float32)


def _sc_degree(dst3):
    @functools.partial(
        pl.kernel,
        out_type=jax.ShapeDtypeStruct((NC, NPAD, 16), jnp.float32),
        mesh=_mesh(),
        compiler_params=pltpu.CompilerParams(use_tc_tiling_on_sc=False),
        scratch_types=[
            pltpu.VMEM((CPT // 2, CHUNK), jnp.int32),
            pltpu.VMEM((CHUNK, 16), jnp.float32),
            pltpu.VMEM((CHUNK, 16), jnp.float32),
            pltpu.VMEM_SHARED((NPAD, 16), jnp.float32),
        ],
    )
    def k(dst_hbm, out_hbm, dst_v, ones_v, zero_v, acc_sh):
        cid = lax.axis_index("c")
        sid = lax.axis_index("s")
        wid = cid * NS + sid

        @pl.loop(0, CHUNK)
        def _(r):
            ones_v[r, pl.ds(0, 16)] = jnp.ones((16,), jnp.float32)
            zero_v[r, pl.ds(0, 16)] = jnp.zeros((16,), jnp.float32)

        @pl.loop(0, 4)
        def _(t):
            pltpu.sync_copy(zero_v, acc_sh.at[pl.ds(sid * RPT + t * CHUNK, CHUNK)])
        pltpu.sync_copy(zero_v.at[pl.ds(0, RPT - 4 * CHUNK)],
                        acc_sh.at[pl.ds(sid * RPT + 4 * CHUNK, RPT - 4 * CHUNK)])
        plsc.subcore_barrier()

        pltpu.sync_copy(dst_hbm.at[wid], dst_v)

        @pl.loop(0, CPT // 2)
        def _(c):
            pltpu.sync_copy(ones_v, acc_sh.at[dst_v.at[c]], add=True)

        plsc.subcore_barrier()
        pltpu.sync_copy(acc_sh.at[pl.ds(sid * RPT, RPT)],
                        out_hbm.at[cid].at[pl.ds(sid * RPT, RPT)])

    return k(dst3)


def _sc_aggregate(gsplit, src3, dst3):
    @functools.partial(
        pl.kernel,
        out_type=jax.ShapeDtypeStruct((NC, NPAD, DH), jnp.float32),
        mesh=_mesh(),
        compiler_params=pltpu.CompilerParams(use_tc_tiling_on_sc=False),
        scratch_types=[
            pltpu.VMEM((CPT, CHUNK), jnp.int32),
            pltpu.VMEM((CPT, CHUNK), jnp.int32),
            pltpu.VMEM((CHUNK, DH), jnp.float32),
            pltpu.VMEM((CHUNK, DH), jnp.float32),
            pltpu.VMEM((CHUNK, DH), jnp.float32),
            pltpu.VMEM((CHUNK, DH), jnp.float32),
            pltpu.VMEM((CHUNK, DH), jnp.float32),
            pltpu.VMEM_SHARED((NPAD, DH), jnp.float32),
            pltpu.SemaphoreType.DMA,
            pltpu.SemaphoreType.DMA,
            pltpu.SemaphoreType.DMA,
            pltpu.SemaphoreType.DMA,
            pltpu.SemaphoreType.DMA,
            pltpu.SemaphoreType.DMA,
            pltpu.SemaphoreType.DMA,
            pltpu.SemaphoreType.DMA,
        ],
    )
    def k(g_hbm, src_hbm, dst_hbm, out_hbm, src_v, dst_v, rows0, rows1,
          rows2, rows3, zero_v, acc_sh, ga, gb, gc, gd, sa, sb, sc_, sd):
        cid = lax.axis_index("c")
        sid = lax.axis_index("s")

        pltpu.sync_copy(src_hbm.at[sid], src_v)
        pltpu.sync_copy(dst_hbm.at[sid], dst_v)

        _zero_rows(zero_v, CHUNK, DH)

        @pl.loop(0, 4)
        def _(t):
            pltpu.sync_copy(zero_v, acc_sh.at[pl.ds(sid * RPT + t * CHUNK, CHUNK)])
        pltpu.sync_copy(zero_v.at[pl.ds(0, RPT - 4 * CHUNK)],
                        acc_sh.at[pl.ds(sid * RPT + 4 * CHUNK, RPT - 4 * CHUNK)])
        plsc.subcore_barrier()

        gsrc = g_hbm.at[cid]
        rows = [rows0, rows1, rows2, rows3]
        semg = [ga, gb, gc, gd]
        sems = [sa, sb, sc_, sd]
        pltpu.async_copy(gsrc.at[src_v.at[0]], rows[0], semg[0])
        pltpu.async_copy(gsrc.at[src_v.at[1]], rows[1], semg[1])
        nq = CPT // 4

        @pl.loop(0, nq)
        def _(q):
            for j in range(4):
                c = 4 * q + j
                j2 = (j + 2) % 4
                pltpu.make_async_copy(gsrc.at[src_v.at[0]], rows[j],
                                      semg[j]).wait()
                pltpu.async_copy(rows[j], acc_sh.at[dst_v.at[c]],
                                 sems[j], add=True)
                if j < 2:
                    @pl.when(q > 0)
                    def _():
                        pltpu.make_async_copy(rows[j2], acc_sh.at[dst_v.at[0]],
                                              sems[j2]).wait()
                    pltpu.async_copy(gsrc.at[src_v.at[c + 2]], rows[j2],
                                     semg[j2])
                else:
                    pltpu.make_async_copy(rows[j2], acc_sh.at[dst_v.at[0]],
                                          sems[j2]).wait()

                    @pl.when(q < nq - 1)
                    def _():
                        pltpu.async_copy(gsrc.at[src_v.at[c + 2]], rows[j2],
                                         semg[j2])

        for j in (2, 3):
            pltpu.make_async_copy(rows[j], acc_sh.at[dst_v.at[0]],
                                  sems[j]).wait()

        plsc.subcore_barrier()
        pltpu.sync_copy(acc_sh.at[pl.ds(sid * RPT, RPT)],
                        out_hbm.at[cid].at[pl.ds(sid * RPT, RPT)])

    return k(gsplit, src3, dst3)


_DOT = (((1,), (0,)), ((), ()))


def _dinv_of(d_ref):
    dd = d_ref[...]
    return lax.rsqrt(dd[0, :, 0:1] + dd[1, :, 0:1] + 1.0)


def _tc_layer1(x, deg2, W1s):
    def body(x_ref, d_ref, w_ref, g_ref):
        dinv = _dinv_of(d_ref)
        xx = x_ref[...]
        ww = w_ref[...]
        h0 = lax.dot_general(xx, ww[0], _DOT, precision=lax.Precision.HIGHEST)
        h1 = lax.dot_general(xx, ww[1], _DOT, precision=lax.Precision.HIGHEST)
        g_ref[...] = jnp.stack([h0 * dinv, h1 * dinv])

    return pl.pallas_call(
        body,
        grid=(NBLK,),
        in_specs=[
            pl.BlockSpec((ROWBLK, D), lambda i: (i, 0)),
            pl.BlockSpec((NC, ROWBLK, 16), lambda i: (0, i, 0)),
            pl.BlockSpec((NC, D, DH), lambda i: (0, 0, 0)),
        ],
        out_specs=pl.BlockSpec((NC, ROWBLK, DH), lambda i: (0, i, 0)),
        out_shape=jax.ShapeDtypeStruct((NC, N_NODES, DH), jnp.float32),
    )(x, deg2, W1s)


def _relu_halves(a_ref, g_ref, dinv, b_ref):
    aa = a_ref[...]
    gg = g_ref[...]
    bb = b_ref[...]
    h0 = jnp.maximum(dinv * (aa[0] + gg[0]) + bb[0], 0.0)
    h1 = jnp.maximum(dinv * (aa[1] + gg[1]) + bb[1], 0.0)
    return h0, h1


def _tc_layer2(acc, g1, deg2, b1s, W2t):
    def body(a_ref, g_ref, d_ref, b_ref, w_ref, o_ref):
        dinv = _dinv_of(d_ref)
        h0, h1 = _relu_halves(a_ref, g_ref, dinv, b_ref)
        ww = w_ref[...]
        hp = lax.Precision.HIGHEST
        h2_0 = (lax.dot_general(h0, ww[0, 0], _DOT, precision=hp)
                + lax.dot_general(h1, ww[1, 0], _DOT, precision=hp))
        h2_1 = (lax.dot_general(h0, ww[0, 1], _DOT, precision=hp)
                + lax.dot_general(h1, ww[1, 1], _DOT, precision=hp))
        o_ref[...] = jnp.stack([h2_0 * dinv, h2_1 * dinv])

    return pl.pallas_call(
        body,
        grid=(NBLK,),
        in_specs=[
            pl.BlockSpec((NC, ROWBLK, DH), lambda i: (0, i, 0)),
            pl.BlockSpec((NC, ROWBLK, DH), lambda i: (0, i, 0)),
            pl.BlockSpec((NC, ROWBLK, 16), lambda i: (0, i, 0)),
            pl.BlockSpec((NC, 1, DH), lambda i: (0, 0, 0)),
            pl.BlockSpec((2, 2, DH, DH), lambda i: (0, 0, 0, 0)),
        ],
        out_specs=pl.BlockSpec((NC, ROWBLK, DH), lambda i: (0, i, 0)),
        out_shape=jax.ShapeDtypeStruct((NC, N_NODES, DH), jnp.float32),
    )(acc, g1, deg2, b1s, W2t)


def _tc_final(acc, g2, deg2, b2s, batf, wlt, bl):
    def body(a_ref, g_ref, d_ref, b_ref, bat_ref, wl_ref,
             bl_ref, o_ref, pool_ref):
        i = pl.program_id(0)
        dinv = _dinv_of(d_ref)
        h0, h1 = _relu_halves(a_ref, g_ref, dinv, b_ref)
        bvec = jnp.reshape(bat_ref[0, 0, :], (1, ROWBLK))
        gids = lax.broadcasted_iota(jnp.int32, (N_GRAPHS, ROWBLK), 0)
        m = (bvec == gids).astype(jnp.float32)
        hp = lax.Precision.HIGHEST
        pm = jnp.stack([lax.dot_general(m, h0, _DOT, precision=hp),
                        lax.dot_general(m, h1, _DOT, precision=hp)])

        @pl.when(i == 0)
        def _():
            pool_ref[...] = pm

        @pl.when(i > 0)
        def _():
            pool_ref[...] += pm

        @pl.when(i == NBLK - 1)
        def _():
            pp = pool_ref[...]
            ww = wl_ref[...]
            o_ref[...] = (lax.dot_general(pp[0], ww[0], _DOT, precision=hp)
                          + lax.dot_general(pp[1], ww[1], _DOT, precision=hp)
                          + bl_ref[...])

    return pl.pallas_call(
        body,
        grid=(NBLK,),
        in_specs=[
            pl.BlockSpec((NC, ROWBLK, DH), lambda i: (0, i, 0)),
            pl.BlockSpec((NC, ROWBLK, DH), lambda i: (0, i, 0)),
            pl.BlockSpec((NC, ROWBLK, 16), lambda i: (0, i, 0)),
            pl.BlockSpec((NC, 1, DH), lambda i: (0, 0, 0)),
            pl.BlockSpec((1, 1, ROWBLK), lambda i: (i, 0, 0)),
            pl.BlockSpec((2, DH, D), lambda i: (0, 0, 0)),
            pl.BlockSpec((1, D), lambda i: (0, 0)),
        ],
        out_specs=pl.BlockSpec((N_GRAPHS, D), lambda i: (0, 0)),
        out_shape=jax.ShapeDtypeStruct((N_GRAPHS, D), jnp.float32),
        scratch_shapes=[pltpu.VMEM((2, N_GRAPHS, DH), jnp.float32)],
    )(acc, g2, deg2, b2s, batf, wlt, bl)


def kernel(x, edge_index, batch, W1, b1, W2, b2, Wlin, blin):
    src = edge_index[0].astype(jnp.int32)
    dst = edge_index[1].astype(jnp.int32)
    pad = E_PAD - N_EDGES
    pad_i = jnp.arange(pad, dtype=jnp.int32)
    src_p = jnp.concatenate([src, pad_i % N_NODES])
    dst_p = jnp.concatenate([dst, N_NODES + pad_i % (NPAD - N_NODES)])
    src3 = src_p.reshape(NS, CPT, CHUNK)
    dst3 = dst_p.reshape(NS, CPT, CHUNK)
    dst3_32 = dst_p.reshape(NC * NS, CPT // 2, CHUNK)

    deg2 = _sc_degree(dst3_32)

    w1s = jnp.stack([W1[:, :DH], W1[:, DH:]])
    w2t = W2.reshape(2, DH, 2, DH).transpose(0, 2, 1, 3)
    b1s = jnp.stack([b1[:DH], b1[DH:]]).reshape(NC, 1, DH)
    b2s = jnp.stack([b2[:DH], b2[DH:]]).reshape(NC, 1, DH)
    wl = jnp.zeros((D, D), jnp.float32).at[:, :N_CLASSES].set(Wlin)
    wlt = wl.reshape(2, DH, D)
    bl = jnp.zeros((1, D), jnp.float32).at[0, :N_CLASSES].set(blin)

    g1 = _tc_layer1(x, deg2, w1s)
    acc1 = _sc_aggregate(g1, src3, dst3)
    g2 = _tc_layer2(acc1, g1, deg2, b1s, w2t)
    acc2 = _sc_aggregate(g2, src3, dst3)

    batf = batch.astype(jnp.int32).reshape(NBLK, 1, ROWBLK)
    outp = _tc_final(acc2, g2, deg2, b2s, batf, wlt, bl)
    return outp[:, :N_CLASSES]


# speedup vs baseline: 2.2758x; 1.0664x over previous
import functools

import jax
import jax.numpy as jnp
from jax import lax
from jax.experimental import pallas as pl
from jax.experimental.pallas import tpu as pltpu
from jax.experimental.pallas import tpu_sc as plsc

N_NODES = 10000
N_EDGES = 320000
D = 128
DH = D // 2
N_GRAPHS = 64
N_CLASSES = 10

NC = 2
NS = 16
CHUNK = 128
CPT = 160
E_PAD = NS * CPT * CHUNK
NPAD = N_NODES + 112
RPT = NPAD // NS
ROWBLK = 1000
NBLK = N_NODES // ROWBLK


def _mesh():
    return plsc.VectorSubcoreMesh(core_axis_name="c", subcore_axis_name="s")


def _zero_rows(ref, nrows, ncols):
    @pl.loop(0, nrows)
    def _(r):
        @pl.loop(0, ncols // 16)
        def _(j):
            ref[r, pl.ds(j * 16, 16)] = jnp.zeros((16,), jnp.float32)


def _sc_degree(dst3):
    @functools.partial(
        pl.kernel,
        out_type=jax.ShapeDtypeStruct((NC, NPAD, 16), jnp.float32),
        mesh=_mesh(),
        compiler_params=pltpu.CompilerParams(use_tc_tiling_on_sc=False),
        scratch_types=[
            pltpu.VMEM((CPT // 2, CHUNK), jnp.int32),
            pltpu.VMEM((CHUNK, 16), jnp.float32),
            pltpu.VMEM((CHUNK, 16), jnp.float32),
            pltpu.VMEM_SHARED((NPAD, 16), jnp.float32),
        ],
    )
    def k(dst_hbm, out_hbm, dst_v, ones_v, zero_v, acc_sh):
        cid = lax.axis_index("c")
        sid = lax.axis_index("s")
        wid = cid * NS + sid

        @pl.loop(0, CHUNK)
        def _(r):
            ones_v[r, pl.ds(0, 16)] = jnp.ones((16,), jnp.float32)
            zero_v[r, pl.ds(0, 16)] = jnp.zeros((16,), jnp.float32)

        @pl.loop(0, 4)
        def _(t):
            pltpu.sync_copy(zero_v, acc_sh.at[pl.ds(sid * RPT + t * CHUNK, CHUNK)])
        pltpu.sync_copy(zero_v.at[pl.ds(0, RPT - 4 * CHUNK)],
                        acc_sh.at[pl.ds(sid * RPT + 4 * CHUNK, RPT - 4 * CHUNK)])
        plsc.subcore_barrier()

        pltpu.sync_copy(dst_hbm.at[wid], dst_v)

        @pl.loop(0, CPT // 2)
        def _(c):
            pltpu.sync_copy(ones_v, acc_sh.at[dst_v.at[c]], add=True)

        plsc.subcore_barrier()
        pltpu.sync_copy(acc_sh.at[pl.ds(sid * RPT, RPT)],
                        out_hbm.at[cid].at[pl.ds(sid * RPT, RPT)])

    return k(dst3)


def _sc_aggregate(gsplit, src3, dst3):
    @functools.partial(
        pl.kernel,
        out_type=jax.ShapeDtypeStruct((NC, NPAD, DH), jnp.float32),
        mesh=_mesh(),
        compiler_params=pltpu.CompilerParams(use_tc_tiling_on_sc=False),
        scratch_types=[
            pltpu.VMEM((CPT, CHUNK), jnp.int32),
            pltpu.VMEM((CPT, CHUNK), jnp.int32),
            pltpu.VMEM((CHUNK, DH), jnp.float32),
            pltpu.VMEM((CHUNK, DH), jnp.float32),
            pltpu.VMEM((CHUNK, DH), jnp.float32),
            pltpu.VMEM((CHUNK, DH), jnp.float32),
            pltpu.VMEM((CHUNK, DH), jnp.float32),
            pltpu.VMEM_SHARED((NPAD, DH), jnp.float32),
            pltpu.SemaphoreType.DMA,
            pltpu.SemaphoreType.DMA,
            pltpu.SemaphoreType.DMA,
            pltpu.SemaphoreType.DMA,
            pltpu.SemaphoreType.DMA,
            pltpu.SemaphoreType.DMA,
            pltpu.SemaphoreType.DMA,
            pltpu.SemaphoreType.DMA,
        ],
    )
    def k(g_hbm, src_hbm, dst_hbm, out_hbm, src_v, dst_v, rows0, rows1,
          rows2, rows3, zero_v, acc_sh, ga, gb, gc, gd, sa, sb, sc_, sd):
        cid = lax.axis_index("c")
        sid = lax.axis_index("s")

        pltpu.sync_copy(src_hbm.at[sid], src_v)
        pltpu.sync_copy(dst_hbm.at[sid], dst_v)

        _zero_rows(zero_v, CHUNK, DH)

        @pl.loop(0, 4)
        def _(t):
            pltpu.sync_copy(zero_v, acc_sh.at[pl.ds(sid * RPT + t * CHUNK, CHUNK)])
        pltpu.sync_copy(zero_v.at[pl.ds(0, RPT - 4 * CHUNK)],
                        acc_sh.at[pl.ds(sid * RPT + 4 * CHUNK, RPT - 4 * CHUNK)])
        plsc.subcore_barrier()

        gsrc = g_hbm.at[cid]
        rows = [rows0, rows1, rows2, rows3]
        semg = [ga, gb, gc, gd]
        sems = [sa, sb, sc_, sd]
        pltpu.async_copy(gsrc.at[src_v.at[0]], rows[0], semg[0])
        pltpu.async_copy(gsrc.at[src_v.at[1]], rows[1], semg[1])
        nq = CPT // 4

        @pl.loop(0, nq)
        def _(q):
            for j in range(4):
                c = 4 * q + j
                j2 = (j + 2) % 4
                pltpu.make_async_copy(gsrc.at[src_v.at[0]], rows[j],
                                      semg[j]).wait()
                pltpu.async_copy(rows[j], acc_sh.at[dst_v.at[c]],
                                 sems[j], add=True)
                if j < 2:
                    @pl.when(q > 0)
                    def _():
                        pltpu.make_async_copy(rows[j2], acc_sh.at[dst_v.at[0]],
                                              sems[j2]).wait()
                    pltpu.async_copy(gsrc.at[src_v.at[c + 2]], rows[j2],
                                     semg[j2])
                else:
                    pltpu.make_async_copy(rows[j2], acc_sh.at[dst_v.at[0]],
                                          sems[j2]).wait()

                    @pl.when(q < nq - 1)
                    def _():
                        pltpu.async_copy(gsrc.at[src_v.at[c + 2]], rows[j2],
                                         semg[j2])

        for j in (2, 3):
            pltpu.make_async_copy(rows[j], acc_sh.at[dst_v.at[0]],
                                  sems[j]).wait()

        plsc.subcore_barrier()
        pltpu.sync_copy(acc_sh.at[pl.ds(sid * RPT, RPT)],
                        out_hbm.at[cid].at[pl.ds(sid * RPT, RPT)])

    return k(gsplit, src3, dst3)


_DOT = (((1,), (0,)), ((), ()))


def _dinv_of(d_ref):
    dd = d_ref[...]
    return lax.rsqrt(dd[0, :, 0:1] + dd[1, :, 0:1] + 1.0)


def _acc_full(a_ref, g_ref):
    aa = a_ref[...]
    gg = g_ref[...]
    return jnp.concatenate([aa[0] + gg[0], aa[1] + gg[1]], axis=-1)


def _tc_layer1(x, deg2, W1):
    def body(x_ref, d_ref, w_ref, g_ref):
        dinv = _dinv_of(d_ref)
        h = lax.dot_general(x_ref[...], w_ref[...], _DOT,
                            precision=lax.Precision.HIGHEST)
        g = h * dinv
        g_ref[...] = jnp.stack([g[:, :DH], g[:, DH:]])

    return pl.pallas_call(
        body,
        grid=(NBLK,),
        in_specs=[
            pl.BlockSpec((ROWBLK, D), lambda i: (i, 0)),
            pl.BlockSpec((NC, ROWBLK, 16), lambda i: (0, i, 0)),
            pl.BlockSpec((D, D), lambda i: (0, 0)),
        ],
        out_specs=pl.BlockSpec((NC, ROWBLK, DH), lambda i: (0, i, 0)),
        out_shape=jax.ShapeDtypeStruct((NC, N_NODES, DH), jnp.float32),
    )(x, deg2, W1)


def _tc_layer2(acc, g1, deg2, b1, W2):
    def body(a_ref, g_ref, d_ref, b_ref, w_ref, o_ref):
        dinv = _dinv_of(d_ref)
        z = dinv * _acc_full(a_ref, g_ref) + b_ref[...]
        h = jnp.maximum(z, 0.0)
        h2 = lax.dot_general(h, w_ref[...], _DOT,
                             precision=lax.Precision.HIGHEST)
        g = h2 * dinv
        o_ref[...] = jnp.stack([g[:, :DH], g[:, DH:]])

    return pl.pallas_call(
        body,
        grid=(NBLK,),
        in_specs=[
            pl.BlockSpec((NC, ROWBLK, DH), lambda i: (0, i, 0)),
            pl.BlockSpec((NC, ROWBLK, DH), lambda i: (0, i, 0)),
            pl.BlockSpec((NC, ROWBLK, 16), lambda i: (0, i, 0)),
            pl.BlockSpec((1, D), lambda i: (0, 0)),
            pl.BlockSpec((D, D), lambda i: (0, 0)),
        ],
        out_specs=pl.BlockSpec((NC, ROWBLK, DH), lambda i: (0, i, 0)),
        out_shape=jax.ShapeDtypeStruct((NC, N_NODES, DH), jnp.float32),
    )(acc, g1, deg2, b1, W2)


def _tc_final(acc, g2, deg2, b2, batf, wl, bl):
    def body(a_ref, g_ref, d_ref, b_ref, bat_ref, wl_ref,
             bl_ref, o_ref, pool_ref):
        i = pl.program_id(0)
        dinv = _dinv_of(d_ref)
        z = dinv * _acc_full(a_ref, g_ref) + b_ref[...]
        h = jnp.maximum(z, 0.0)
        bvec = jnp.reshape(bat_ref[0, 0, :], (1, ROWBLK))
        gids = lax.broadcasted_iota(jnp.int32, (N_GRAPHS, ROWBLK), 0)
        m = (bvec == gids).astype(jnp.float32)
        pm = lax.dot_general(m, h, _DOT, precision=lax.Precision.HIGHEST)

        @pl.when(i == 0)
        def _():
            pool_ref[...] = pm

        @pl.when(i > 0)
        def _():
            pool_ref[...] += pm

        @pl.when(i == NBLK - 1)
        def _():
            o_ref[...] = lax.dot_general(pool_ref[...], wl_ref[...], _DOT,
                                         precision=lax.Precision.HIGHEST) + bl_ref[...]

    return pl.pallas_call(
        body,
        grid=(NBLK,),
        in_specs=[
            pl.BlockSpec((NC, ROWBLK, DH), lambda i: (0, i, 0)),
            pl.BlockSpec((NC, ROWBLK, DH), lambda i: (0, i, 0)),
            pl.BlockSpec((NC, ROWBLK, 16), lambda i: (0, i, 0)),
            pl.BlockSpec((1, D), lambda i: (0, 0)),
            pl.BlockSpec((1, 1, ROWBLK), lambda i: (i, 0, 0)),
            pl.BlockSpec((D, D), lambda i: (0, 0)),
            pl.BlockSpec((1, D), lambda i: (0, 0)),
        ],
        out_specs=pl.BlockSpec((N_GRAPHS, D), lambda i: (0, 0)),
        out_shape=jax.ShapeDtypeStruct((N_GRAPHS, D), jnp.float32),
        scratch_shapes=[pltpu.VMEM((N_GRAPHS, D), jnp.float32)],
    )(acc, g2, deg2, b2, batf, wl, bl)


def kernel(x, edge_index, batch, W1, b1, W2, b2, Wlin, blin):
    src = edge_index[0].astype(jnp.int32)
    dst = edge_index[1].astype(jnp.int32)
    pad = E_PAD - N_EDGES
    pad_i = jnp.arange(pad, dtype=jnp.int32)
    src_p = jnp.concatenate([src, pad_i % N_NODES])
    dst_p = jnp.concatenate([dst, N_NODES + pad_i % (NPAD - N_NODES)])
    src3 = src_p.reshape(NS, CPT, CHUNK)
    dst3 = dst_p.reshape(NS, CPT, CHUNK)
    dst3_32 = dst_p.reshape(NC * NS, CPT // 2, CHUNK)

    deg2 = _sc_degree(dst3_32)

    wl = jnp.zeros((D, D), jnp.float32).at[:, :N_CLASSES].set(Wlin)
    bl = jnp.zeros((1, D), jnp.float32).at[0, :N_CLASSES].set(blin)

    g1 = _tc_layer1(x, deg2, W1)
    acc1 = _sc_aggregate(g1, src3, dst3)
    g2 = _tc_layer2(acc1, g1, deg2, jnp.reshape(b1, (1, D)), W2)
    acc2 = _sc_aggregate(g2, src3, dst3)

    batf = batch.astype(jnp.int32).reshape(NBLK, 1, ROWBLK)
    outp = _tc_final(acc2, g2, deg2, jnp.reshape(b2, (1, D)), batf, wl, bl)
    return outp[:, :N_CLASSES]


# speedup vs baseline: 2.2979x; 1.0097x over previous
import functools

import jax
import jax.numpy as jnp
from jax import lax
from jax.experimental import pallas as pl
from jax.experimental.pallas import tpu as pltpu
from jax.experimental.pallas import tpu_sc as plsc

N_NODES = 10000
N_EDGES = 320000
D = 128
DH = D // 2
N_GRAPHS = 64
N_CLASSES = 10

NC = 2
NS = 16
CHUNK = 128
CPT = 160
E_PAD = NS * CPT * CHUNK
NPAD = N_NODES + 112
RPT = NPAD // NS
ROWBLK = 1000
NBLK = N_NODES // ROWBLK


def _mesh():
    return plsc.VectorSubcoreMesh(core_axis_name="c", subcore_axis_name="s")


def _zero_rows(ref, nrows, ncols):
    @pl.loop(0, nrows)
    def _(r):
        @pl.loop(0, ncols // 16)
        def _(j):
            ref[r, pl.ds(j * 16, 16)] = jnp.zeros((16,), jnp.float32)


def _sc_degree(dst3):
    @functools.partial(
        pl.kernel,
        out_type=jax.ShapeDtypeStruct((NC, NPAD, 16), jnp.float32),
        mesh=_mesh(),
        compiler_params=pltpu.CompilerParams(use_tc_tiling_on_sc=False),
        scratch_types=[
            pltpu.VMEM((CPT // 2, CHUNK), jnp.int32),
            pltpu.VMEM((CHUNK, 16), jnp.float32),
            pltpu.VMEM((CHUNK, 16), jnp.float32),
            pltpu.VMEM_SHARED((NPAD, 16), jnp.float32),
            pltpu.SemaphoreType.DMA,
        ],
    )
    def k(dst_hbm, out_hbm, dst_v, ones_v, zero_v, acc_sh, semd):
        cid = lax.axis_index("c")
        sid = lax.axis_index("s")
        wid = cid * NS + sid

        @pl.loop(0, CHUNK)
        def _(r):
            ones_v[r, pl.ds(0, 16)] = jnp.ones((16,), jnp.float32)
            zero_v[r, pl.ds(0, 16)] = jnp.zeros((16,), jnp.float32)

        @pl.loop(0, 4)
        def _(t):
            pltpu.sync_copy(zero_v, acc_sh.at[pl.ds(sid * RPT + t * CHUNK, CHUNK)])
        pltpu.sync_copy(zero_v.at[pl.ds(0, RPT - 4 * CHUNK)],
                        acc_sh.at[pl.ds(sid * RPT + 4 * CHUNK, RPT - 4 * CHUNK)])
        plsc.subcore_barrier()

        pltpu.sync_copy(dst_hbm.at[wid], dst_v)

        @pl.loop(0, CPT // 2)
        def _(c):
            pltpu.async_copy(ones_v, acc_sh.at[dst_v.at[c]], semd, add=True)

        @pl.loop(0, CPT // 2)
        def _(c):
            pltpu.make_async_copy(ones_v, acc_sh.at[dst_v.at[0]], semd).wait()

        plsc.subcore_barrier()
        pltpu.sync_copy(acc_sh.at[pl.ds(sid * RPT, RPT)],
                        out_hbm.at[cid].at[pl.ds(sid * RPT, RPT)])

    return k(dst3)


def _sc_aggregate(gsplit, src3, dst3):
    @functools.partial(
        pl.kernel,
        out_type=jax.ShapeDtypeStruct((NC, NPAD, DH), jnp.float32),
        mesh=_mesh(),
        compiler_params=pltpu.CompilerParams(use_tc_tiling_on_sc=False),
        scratch_types=[
            pltpu.VMEM((CPT, CHUNK), jnp.int32),
            pltpu.VMEM((CPT, CHUNK), jnp.int32),
            pltpu.VMEM((CHUNK, DH), jnp.float32),
            pltpu.VMEM((CHUNK, DH), jnp.float32),
            pltpu.VMEM((CHUNK, DH), jnp.float32),
            pltpu.VMEM((CHUNK, DH), jnp.float32),
            pltpu.VMEM((CHUNK, DH), jnp.float32),
            pltpu.VMEM_SHARED((NPAD, DH), jnp.float32),
            pltpu.SemaphoreType.DMA,
            pltpu.SemaphoreType.DMA,
            pltpu.SemaphoreType.DMA,
            pltpu.SemaphoreType.DMA,
            pltpu.SemaphoreType.DMA,
            pltpu.SemaphoreType.DMA,
            pltpu.SemaphoreType.DMA,
            pltpu.SemaphoreType.DMA,
        ],
    )
    def k(g_hbm, src_hbm, dst_hbm, out_hbm, src_v, dst_v, rows0, rows1,
          rows2, rows3, zero_v, acc_sh, ga, gb, gc, gd, sa, sb, sc_, sd):
        cid = lax.axis_index("c")
        sid = lax.axis_index("s")

        pltpu.sync_copy(src_hbm.at[sid], src_v)
        pltpu.sync_copy(dst_hbm.at[sid], dst_v)

        _zero_rows(zero_v, CHUNK, DH)

        @pl.loop(0, 4)
        def _(t):
            pltpu.sync_copy(zero_v, acc_sh.at[pl.ds(sid * RPT + t * CHUNK, CHUNK)])
        pltpu.sync_copy(zero_v.at[pl.ds(0, RPT - 4 * CHUNK)],
                        acc_sh.at[pl.ds(sid * RPT + 4 * CHUNK, RPT - 4 * CHUNK)])
        plsc.subcore_barrier()

        gsrc = g_hbm.at[cid]
        rows = [rows0, rows1, rows2, rows3]
        semg = [ga, gb, gc, gd]
        sems = [sa, sb, sc_, sd]
        pltpu.async_copy(gsrc.at[src_v.at[0]], rows[0], semg[0])
        pltpu.async_copy(gsrc.at[src_v.at[1]], rows[1], semg[1])
        nq = CPT // 4

        @pl.loop(0, nq)
        def _(q):
            for j in range(4):
                c = 4 * q + j
                j2 = (j + 2) % 4
                pltpu.make_async_copy(gsrc.at[src_v.at[0]], rows[j],
                                      semg[j]).wait()
                pltpu.async_copy(rows[j], acc_sh.at[dst_v.at[c]],
                                 sems[j], add=True)
                if j < 2:
                    @pl.when(q > 0)
                    def _():
                        pltpu.make_async_copy(rows[j2], acc_sh.at[dst_v.at[0]],
                                              sems[j2]).wait()
                    pltpu.async_copy(gsrc.at[src_v.at[c + 2]], rows[j2],
                                     semg[j2])
                else:
                    pltpu.make_async_copy(rows[j2], acc_sh.at[dst_v.at[0]],
                                          sems[j2]).wait()

                    @pl.when(q < nq - 1)
                    def _():
                        pltpu.async_copy(gsrc.at[src_v.at[c + 2]], rows[j2],
                                         semg[j2])

        for j in (2, 3):
            pltpu.make_async_copy(rows[j], acc_sh.at[dst_v.at[0]],
                                  sems[j]).wait()

        plsc.subcore_barrier()
        pltpu.sync_copy(acc_sh.at[pl.ds(sid * RPT, RPT)],
                        out_hbm.at[cid].at[pl.ds(sid * RPT, RPT)])

    return k(gsplit, src3, dst3)


_DOT = (((1,), (0,)), ((), ()))


def _dinv_of(d_ref):
    dd = d_ref[...]
    return lax.rsqrt(dd[0, :, 0:1] + dd[1, :, 0:1] + 1.0)


def _acc_full(a_ref, g_ref):
    aa = a_ref[...]
    gg = g_ref[...]
    return jnp.concatenate([aa[0] + gg[0], aa[1] + gg[1]], axis=-1)


def _tc_layer1(x, deg2, W1):
    def body(x_ref, d_ref, w_ref, g_ref):
        dinv = _dinv_of(d_ref)
        h = lax.dot_general(x_ref[...], w_ref[...], _DOT,
                            precision=lax.Precision.HIGHEST)
        g = h * dinv
        g_ref[...] = jnp.stack([g[:, :DH], g[:, DH:]])

    return pl.pallas_call(
        body,
        grid=(NBLK,),
        in_specs=[
            pl.BlockSpec((ROWBLK, D), lambda i: (i, 0)),
            pl.BlockSpec((NC, ROWBLK, 16), lambda i: (0, i, 0)),
            pl.BlockSpec((D, D), lambda i: (0, 0)),
        ],
        out_specs=pl.BlockSpec((NC, ROWBLK, DH), lambda i: (0, i, 0)),
        out_shape=jax.ShapeDtypeStruct((NC, N_NODES, DH), jnp.float32),
    )(x, deg2, W1)


def _tc_layer2(acc, g1, deg2, b1, W2):
    def body(a_ref, g_ref, d_ref, b_ref, w_ref, o_ref):
        dinv = _dinv_of(d_ref)
        z = dinv * _acc_full(a_ref, g_ref) + b_ref[...]
        h = jnp.maximum(z, 0.0)
        h2 = lax.dot_general(h, w_ref[...], _DOT,
                             precision=lax.Precision.HIGHEST)
        g = h2 * dinv
        o_ref[...] = jnp.stack([g[:, :DH], g[:, DH:]])

    return pl.pallas_call(
        body,
        grid=(NBLK,),
        in_specs=[
            pl.BlockSpec((NC, ROWBLK, DH), lambda i: (0, i, 0)),
            pl.BlockSpec((NC, ROWBLK, DH), lambda i: (0, i, 0)),
            pl.BlockSpec((NC, ROWBLK, 16), lambda i: (0, i, 0)),
            pl.BlockSpec((1, D), lambda i: (0, 0)),
            pl.BlockSpec((D, D), lambda i: (0, 0)),
        ],
        out_specs=pl.BlockSpec((NC, ROWBLK, DH), lambda i: (0, i, 0)),
        out_shape=jax.ShapeDtypeStruct((NC, N_NODES, DH), jnp.float32),
    )(acc, g1, deg2, b1, W2)


def _tc_final(acc, g2, deg2, b2, batf, wl, bl):
    def body(a_ref, g_ref, d_ref, b_ref, bat_ref, wl_ref,
             bl_ref, o_ref, pool_ref):
        i = pl.program_id(0)
        dinv = _dinv_of(d_ref)
        z = dinv * _acc_full(a_ref, g_ref) + b_ref[...]
        h = jnp.maximum(z, 0.0)
        bvec = jnp.reshape(bat_ref[0, 0, :], (1, ROWBLK))
        gids = lax.broadcasted_iota(jnp.int32, (N_GRAPHS, ROWBLK), 0)
        m = (bvec == gids).astype(jnp.float32)
        pm = lax.dot_general(m, h, _DOT, precision=lax.Precision.HIGHEST)

        @pl.when(i == 0)
        def _():
            pool_ref[...] = pm

        @pl.when(i > 0)
        def _():
            pool_ref[...] += pm

        @pl.when(i == NBLK - 1)
        def _():
            o_ref[...] = lax.dot_general(pool_ref[...], wl_ref[...], _DOT,
                                         precision=lax.Precision.HIGHEST) + bl_ref[...]

    return pl.pallas_call(
        body,
        grid=(NBLK,),
        in_specs=[
            pl.BlockSpec((NC, ROWBLK, DH), lambda i: (0, i, 0)),
            pl.BlockSpec((NC, ROWBLK, DH), lambda i: (0, i, 0)),
            pl.BlockSpec((NC, ROWBLK, 16), lambda i: (0, i, 0)),
            pl.BlockSpec((1, D), lambda i: (0, 0)),
            pl.BlockSpec((1, 1, ROWBLK), lambda i: (i, 0, 0)),
            pl.BlockSpec((D, D), lambda i: (0, 0)),
            pl.BlockSpec((1, D), lambda i: (0, 0)),
        ],
        out_specs=pl.BlockSpec((N_GRAPHS, D), lambda i: (0, 0)),
        out_shape=jax.ShapeDtypeStruct((N_GRAPHS, D), jnp.float32),
        scratch_shapes=[pltpu.VMEM((N_GRAPHS, D), jnp.float32)],
    )(acc, g2, deg2, b2, batf, wl, bl)


def kernel(x, edge_index, batch, W1, b1, W2, b2, Wlin, blin):
    src = edge_index[0].astype(jnp.int32)
    dst = edge_index[1].astype(jnp.int32)
    pad = E_PAD - N_EDGES
    pad_i = jnp.arange(pad, dtype=jnp.int32)
    src_p = jnp.concatenate([src, pad_i % N_NODES])
    dst_p = jnp.concatenate([dst, N_NODES + pad_i % (NPAD - N_NODES)])
    src3 = src_p.reshape(NS, CPT, CHUNK)
    dst3 = dst_p.reshape(NS, CPT, CHUNK)
    dst3_32 = dst_p.reshape(NC * NS, CPT // 2, CHUNK)

    deg2 = _sc_degree(dst3_32)

    wl = jnp.zeros((D, D), jnp.float32).at[:, :N_CLASSES].set(Wlin)
    bl = jnp.zeros((1, D), jnp.float32).at[0, :N_CLASSES].set(blin)

    g1 = _tc_layer1(x, deg2, W1)
    acc1 = _sc_aggregate(g1, src3, dst3)
    g2 = _tc_layer2(acc1, g1, deg2, jnp.reshape(b1, (1, D)), W2)
    acc2 = _sc_aggregate(g2, src3, dst3)

    batf = batch.astype(jnp.int32).reshape(NBLK, 1, ROWBLK)
    outp = _tc_final(acc2, g2, deg2, jnp.reshape(b2, (1, D)), batf, wl, bl)
    return outp[:, :N_CLASSES]
